# baseline (speedup 1.0000x reference)
import functools

import jax
import jax.numpy as jnp
from jax.experimental import pallas as pl
from jax.experimental.pallas import tpu as pltpu

_B = 16384
_V = 1000
_VP = 1024
_D = 128
_BB = 1024


def _tc_body(ids_ref, table_ref, out_ref):
  ids = ids_ref[...]
  iota = jax.lax.broadcasted_iota(jnp.int32, (_BB, _VP), 1)
  oh = (iota == ids).astype(jnp.bfloat16)
  t = table_ref[...]
  t_hi = t.astype(jnp.bfloat16)
  t_lo = (t - t_hi.astype(jnp.float32)).astype(jnp.bfloat16)
  out_ref[...] = (
      jnp.dot(oh, t_hi, preferred_element_type=jnp.float32)
      + jnp.dot(oh, t_lo, preferred_element_type=jnp.float32))


_tc_gather = pl.pallas_call(
    _tc_body,
    grid=(_B // _BB,),
    in_specs=[
        pl.BlockSpec((_BB, 1), lambda i: (i, 0)),
        pl.BlockSpec((_VP, _D), lambda i: (0, 0)),
    ],
    out_specs=pl.BlockSpec((_BB, _D), lambda i: (i, 0)),
    out_shape=jax.ShapeDtypeStruct((_B, _D), jnp.float32),
)


@jax.jit
def kernel(violation_ids, violation_embedding):
  ids2d = violation_ids.astype(jnp.int32).reshape(_B, 1)
  table = jnp.pad(violation_embedding, ((0, _VP - _V), (0, 0)))
  return _tc_gather(ids2d, table)
